# baseline (speedup 1.0000x reference)
import dataclasses
import functools

import jax
import jax.numpy as jnp
from jax import lax
from jax.experimental import pallas as pl
from jax.experimental.pallas import tpu as pltpu
from jax.experimental.pallas import tpu_sc as plsc

B, S, D = 4, 4096, 1024
NS = 256
K = 8
NB = S // K
NSCALE = 9
PROWS = 1024
OFF = [PROWS - (PROWS >> j) for j in range(NSCALE)]
N16 = NB // 16
NL16 = 5
TROWS = PROWS + NL16 * N16
L = 16
NV = D // L
MAXROWS = 32
EB1, EB2 = 16, 24
NW = 32
SPW = (B * NS) // NW
NEG = float(jnp.finfo(jnp.float32).min)
DH = 512


def _table_body(x_ref, t_ref, z_ref):
    z_ref[0] = jnp.zeros((S, DH), jnp.float32)
    x = x_ref[0]
    cur = jnp.max(x.reshape(NB, K, DH), axis=1)
    t_ref[0, 0:NB] = cur
    lat = None
    for j in range(1, NSCALE):
        n = NB >> j
        cur = jnp.max(cur.reshape(n, 2, DH), axis=1)
        t_ref[0, OFF[j]:OFF[j] + n] = cur
        if j == 4:
            lat = cur
    t_ref[0, PROWS - 2:PROWS] = cur
    for l in range(1, NL16 + 1):
        h = 1 << (l - 1)
        lat = jnp.maximum(lat, jnp.concatenate([lat[h:], lat[:h]], axis=0))
        o = PROWS + (l - 1) * N16
        t_ref[0, o:o + N16] = lat


def _build_table(context):
    return pl.pallas_call(
        _table_body,
        grid=(B, D // DH),
        in_specs=[pl.BlockSpec((1, S, DH), lambda b, d: (b, 0, d))],
        out_specs=[pl.BlockSpec((1, TROWS, DH), lambda b, d: (b, 0, d)),
                   pl.BlockSpec((1, S, DH), lambda b, d: (b, 0, d))],
        out_shape=[jax.ShapeDtypeStruct((B, TROWS, D), jnp.float32),
                   jax.ShapeDtypeStruct((B, S, D), jnp.float32)],
    )(context)


def _sc_pool(context, table, starts, ends):
    mesh = plsc.VectorSubcoreMesh(core_axis_name="c", subcore_axis_name="s")
    cp = pltpu.CompilerParams()
    if "needs_layout_passes" in pltpu.CompilerParams.__dataclass_fields__:
        cp = dataclasses.replace(cp, needs_layout_passes=False)

    @functools.partial(
        pl.kernel,
        out_type=jax.ShapeDtypeStruct((B, NS, D), jnp.float32),
        mesh=mesh,
        compiler_params=cp,
        scratch_types=[
            pltpu.VMEM((SPW,), jnp.int32),
            pltpu.VMEM((SPW,), jnp.int32),
            pltpu.VMEM((2, MAXROWS, D), jnp.float32),
            pltpu.VMEM((2, D), jnp.float32),
            pltpu.SemaphoreType.DMA,
            pltpu.SemaphoreType.DMA,
            pltpu.SemaphoreType.DMA,
            pltpu.SemaphoreType.DMA,
        ],
    )
    def pool(ctx_hbm, tab_hbm, st_hbm, en_hbm, out_hbm,
             st_v, en_v, rows_v, acc_v, sem_a, sem_b, sem_wa, sem_wb):
        wid = lax.axis_index("s") * 2 + lax.axis_index("c")
        base = wid * SPW
        b = base // NS
        r0 = base % NS
        pltpu.sync_copy(st_hbm.at[pl.ds(base, SPW)], st_v)
        pltpu.sync_copy(en_hbm.at[pl.ds(base, SPW)], en_v)
        lanes = lax.iota(jnp.int32, L)
        neg_vec = jnp.full((L,), NEG, jnp.float32)
        zero_vec = jnp.zeros((L,), jnp.float32)

        def get(vref, j):
            v = jnp.where(j >= L, vref[pl.ds(L, L)], vref[pl.ds(0, L)])
            return jnp.max(jnp.where(lanes == j % L, v, 0))

        def fire_span(j, slot, sem):
            s = get(st_v, j)
            e = get(en_v, j)
            a = (s + K - 1) // K
            bb = e // K
            c = jnp.zeros((), jnp.int32)

            def fire_tab(row, cond, c):
                @pl.when(cond)
                def _():
                    pltpu.async_copy(tab_hbm.at[b, row],
                                     rows_v.at[slot, c], sem)
                return jnp.where(cond, c + 1, c)

            for jj in range(4):
                w = 1 << jj
                up = ((a & w) != 0) & (a < bb)
                c = fire_tab(OFF[jj] + (a >> jj), up, c)
                a = jnp.where(up, a + w, a)
                dn = ((bb & w) != 0) & (a < bb)
                c = fire_tab(OFF[jj] + ((bb - w) >> jj), dn, c)
                bb = jnp.where(dn, bb - w, bb)
            nb16 = (bb - a) >> 4
            l16 = lax.while_loop(lambda l_: (2 << l_) <= nb16,
                                 lambda l_: l_ + 1, 0)
            w16 = 1 << l16
            i1 = a >> 4
            i2 = (bb >> 4) - w16
            row1 = jnp.where(l16 == 0, OFF[4] + i1,
                             PROWS + (l16 - 1) * N16 + i1)
            row2 = jnp.where(l16 == 0, OFF[4] + i2,
                             PROWS + (l16 - 1) * N16 + i2)
            c = fire_tab(row1, nb16 > 0, c)
            c = fire_tab(row2, nb16 > 0, c)

            nonempty = e > s
            bs1 = pl.multiple_of((s >> 3) << 3, K)
            bs2 = pl.multiple_of(((e - 1) >> 3) << 3, K)

            @pl.when(nonempty)
            def _():
                pltpu.async_copy(ctx_hbm.at[b, pl.ds(bs1, K)],
                                 rows_v.at[slot, pl.ds(EB1, K)], sem)
                pltpu.async_copy(ctx_hbm.at[b, pl.ds(bs2, K)],
                                 rows_v.at[slot, pl.ds(EB2, K)], sem)
            lo1 = EB1 + (s - bs1)
            hi1 = EB1 + jnp.minimum(e - bs1, K)
            lo2 = EB2 + jnp.maximum(s - bs2, 0)
            hi2 = EB2 + (e - bs2)
            return (c.astype(jnp.int32), nonempty.astype(jnp.int32),
                    lo1.astype(jnp.int32), hi1.astype(jnp.int32),
                    lo2.astype(jnp.int32), hi2.astype(jnp.int32))

        def finish_span(j, slot, sem, sem_w, meta, warm):
            c, nonempty, lo1, hi1, lo2, hi2 = meta

            @pl.when(warm)
            def _():
                pltpu.make_async_copy(ctx_hbm.at[b, 0],
                                      acc_v.at[slot], sem_w).wait()

            units = c + nonempty * 2 * K
            for bit in (16, 8, 4, 2, 1):
                @pl.when((units & bit) != 0)
                def _():
                    pltpu.make_async_copy(ctx_hbm.at[b, pl.ds(0, bit)],
                                          rows_v.at[0, pl.ds(0, bit)],
                                          sem).wait()

            for half in range(2):
                def mk_acc(lo_off):
                    def acc_row(i, regs):
                        return tuple(
                            jnp.maximum(
                                regs[m],
                                rows_v[slot, i + lo_off,
                                       pl.ds((half * (NV // 2) + m) * L, L)])
                            for m in range(NV // 2))
                    return acc_row
                regs = tuple(neg_vec for _ in range(NV // 2))
                regs = lax.fori_loop(0, c, mk_acc(0), regs)
                regs = lax.fori_loop(0, hi1 - lo1, mk_acc(lo1), regs)
                regs = lax.fori_loop(0, hi2 - lo2, mk_acc(lo2), regs)
                for m in range(NV // 2):
                    acc_v[slot, pl.ds((half * (NV // 2) + m) * L, L)] = jnp.where(
                        nonempty > 0, regs[m], zero_vec)
            pltpu.async_copy(acc_v.at[slot], out_hbm.at[b, r0 + j], sem_w)

        zmeta = tuple(jnp.zeros((), jnp.int32) for _ in range(6))
        meta0 = fire_span(0, 0, sem_a)

        def pair_body(jj, meta_a):
            ja = 2 * jj
            warm = jj >= 1
            meta_b = fire_span(ja + 1, 1, sem_b)
            finish_span(ja, 0, sem_a, sem_wa, meta_a, warm)
            meta_next = lax.cond(
                ja + 2 < SPW,
                lambda: fire_span(ja + 2, 0, sem_a),
                lambda: zmeta)
            finish_span(ja + 1, 1, sem_b, sem_wb, meta_b, warm)
            return meta_next

        lax.fori_loop(0, SPW // 2, pair_body, meta0)
        pltpu.make_async_copy(ctx_hbm.at[b, 0], acc_v.at[0], sem_wa).wait()
        pltpu.make_async_copy(ctx_hbm.at[b, 0], acc_v.at[1], sem_wb).wait()

    return pool(context, table, starts, ends)


def kernel(context, spans):
    spans = spans.astype(jnp.int32)
    starts = spans[:, :, 0].reshape(B * NS)
    ends = spans[:, :, 1].reshape(B * NS)
    table, zeros_out = _build_table(context)
    pooled = _sc_pool(context, table, starts, ends)
    return lax.dynamic_update_slice(zeros_out, pooled, (0, 0, 0))


# speedup vs baseline: 1.1421x; 1.1421x over previous
import dataclasses
import functools

import jax
import jax.numpy as jnp
from jax import lax
from jax.experimental import pallas as pl
from jax.experimental.pallas import tpu as pltpu
from jax.experimental.pallas import tpu_sc as plsc

B, S, D = 4, 4096, 1024
NS = 256
K = 8
NB = S // K
NSCALE = 9
PROWS = 1024
OFF = [PROWS - (PROWS >> j) for j in range(NSCALE)]
N16 = NB // 16
NL16 = 5
TROWS = PROWS + NL16 * N16
L = 16
NV = D // L
MAXROWS = 24
NW = 32
SPW = (B * NS) // NW
NEG = float(jnp.finfo(jnp.float32).min)
DH = 512


def _table_body(x_ref, t_ref, z_ref):
    z_ref[0] = jnp.zeros((S, DH), jnp.float32)
    x = x_ref[0]
    cur = jnp.max(x.reshape(NB, K, DH), axis=1)
    t_ref[0, 0:NB] = cur
    lat = None
    for j in range(1, NSCALE):
        n = NB >> j
        cur = jnp.max(cur.reshape(n, 2, DH), axis=1)
        t_ref[0, OFF[j]:OFF[j] + n] = cur
        if j == 4:
            lat = cur
    t_ref[0, PROWS - 2:PROWS] = cur
    for l in range(1, NL16 + 1):
        h = 1 << (l - 1)
        lat = jnp.maximum(lat, jnp.concatenate([lat[h:], lat[:h]], axis=0))
        o = PROWS + (l - 1) * N16
        t_ref[0, o:o + N16] = lat


def _build_table(context):
    return pl.pallas_call(
        _table_body,
        grid=(B, D // DH),
        in_specs=[pl.BlockSpec((1, S, DH), lambda b, d: (b, 0, d))],
        out_specs=[pl.BlockSpec((1, TROWS, DH), lambda b, d: (b, 0, d)),
                   pl.BlockSpec((1, S, DH), lambda b, d: (b, 0, d))],
        out_shape=[jax.ShapeDtypeStruct((B, TROWS, D), jnp.float32),
                   jax.ShapeDtypeStruct((B, S, D), jnp.float32)],
    )(context)


def _sc_pool(context, table, starts, ends):
    mesh = plsc.VectorSubcoreMesh(core_axis_name="c", subcore_axis_name="s")
    cp = pltpu.CompilerParams()
    if "needs_layout_passes" in pltpu.CompilerParams.__dataclass_fields__:
        cp = dataclasses.replace(cp, needs_layout_passes=False)

    @functools.partial(
        pl.kernel,
        out_type=jax.ShapeDtypeStruct((B, NS, D), jnp.float32),
        mesh=mesh,
        compiler_params=cp,
        scratch_types=[
            pltpu.VMEM((SPW,), jnp.int32),
            pltpu.VMEM((SPW,), jnp.int32),
            pltpu.VMEM((2, MAXROWS, D), jnp.float32),
            pltpu.VMEM((D,), jnp.float32),
            pltpu.SemaphoreType.DMA,
            pltpu.SemaphoreType.DMA,
        ],
    )
    def pool(ctx_hbm, tab_hbm, st_hbm, en_hbm, out_hbm,
             st_v, en_v, rows_v, acc_v, sem_a, sem_b):
        wid = lax.axis_index("s") * 2 + lax.axis_index("c")
        base = wid * SPW
        b = base // NS
        r0 = base % NS
        pltpu.sync_copy(st_hbm.at[pl.ds(base, SPW)], st_v)
        pltpu.sync_copy(en_hbm.at[pl.ds(base, SPW)], en_v)
        lanes = lax.iota(jnp.int32, L)
        neg_vec = jnp.full((L,), NEG, jnp.float32)
        zero_vec = jnp.zeros((L,), jnp.float32)

        def get(vref, j):
            v = jnp.where(j >= L, vref[pl.ds(L, L)], vref[pl.ds(0, L)])
            return jnp.max(jnp.where(lanes == j % L, v, 0))

        def fire_span(j, slot, sem):
            s = get(st_v, j)
            e = get(en_v, j)
            a = (s + K - 1) // K
            bb = e // K
            c = jnp.zeros((), jnp.int32)

            def fire_tab(row, cond, c):
                @pl.when(cond)
                def _():
                    pltpu.async_copy(tab_hbm.at[b, row],
                                     rows_v.at[slot, c], sem)
                return jnp.where(cond, c + 1, c)

            for jj in range(4):
                w = 1 << jj
                up = ((a & w) != 0) & (a < bb)
                c = fire_tab(OFF[jj] + (a >> jj), up, c)
                a = jnp.where(up, a + w, a)
                dn = ((bb & w) != 0) & (a < bb)
                c = fire_tab(OFF[jj] + ((bb - w) >> jj), dn, c)
                bb = jnp.where(dn, bb - w, bb)
            nb16 = (bb - a) >> 4
            l16 = lax.while_loop(lambda l_: (2 << l_) <= nb16,
                                 lambda l_: l_ + 1, 0)
            w16 = 1 << l16
            i1 = a >> 4
            i2 = (bb >> 4) - w16
            row1 = jnp.where(l16 == 0, OFF[4] + i1,
                             PROWS + (l16 - 1) * N16 + i1)
            row2 = jnp.where(l16 == 0, OFF[4] + i2,
                             PROWS + (l16 - 1) * N16 + i2)
            c = fire_tab(row1, nb16 > 0, c)
            c = fire_tab(row2, nb16 > 0, c)

            s8 = (s + K - 1) // K
            e8 = e // K
            n1 = jnp.minimum(e, s8 * K) - s
            lo2 = jnp.maximum(s, e8 * K)
            n2 = e - lo2

            def fire1(i, cc):
                pltpu.async_copy(ctx_hbm.at[b, s + i],
                                 rows_v.at[slot, c + i], sem)
                return cc
            lax.fori_loop(0, n1, fire1, 0)

            def fire2(i, cc):
                pltpu.async_copy(ctx_hbm.at[b, lo2 + i],
                                 rows_v.at[slot, c + n1 + i], sem)
                return cc
            lax.fori_loop(0, n2, fire2, 0)
            return ((c + n1 + n2).astype(jnp.int32),
                    (e > s).astype(jnp.int32))

        def finish_span(j, slot, sem, meta):
            ntot, nonempty = meta

            def drain(i, cc):
                pltpu.make_async_copy(ctx_hbm.at[b, 0],
                                      rows_v.at[0, 0], sem).wait()
                return cc
            lax.fori_loop(0, ntot, drain, 0)

            for half in range(2):
                def acc_row(i, regs):
                    return tuple(
                        jnp.maximum(regs[m],
                                    rows_v[slot, i,
                                           pl.ds((half * (NV // 2) + m) * L, L)])
                        for m in range(NV // 2))
                regs = lax.fori_loop(0, ntot, acc_row,
                                     tuple(neg_vec for _ in range(NV // 2)))
                for m in range(NV // 2):
                    acc_v[pl.ds((half * (NV // 2) + m) * L, L)] = jnp.where(
                        nonempty > 0, regs[m], zero_vec)
            pltpu.sync_copy(acc_v, out_hbm.at[b, r0 + j])

        meta0 = fire_span(0, 0, sem_a)

        def pair_body(jj, meta_a):
            ja = 2 * jj
            meta_b = fire_span(ja + 1, 1, sem_b)
            finish_span(ja, 0, sem_a, meta_a)
            meta_next = lax.cond(
                ja + 2 < SPW,
                lambda: fire_span(ja + 2, 0, sem_a),
                lambda: (jnp.zeros((), jnp.int32), jnp.zeros((), jnp.int32)))
            finish_span(ja + 1, 1, sem_b, meta_b)
            return meta_next

        lax.fori_loop(0, SPW // 2, pair_body, meta0)

    return pool(context, table, starts, ends)


def kernel(context, spans):
    spans = spans.astype(jnp.int32)
    starts = spans[:, :, 0].reshape(B * NS)
    ends = spans[:, :, 1].reshape(B * NS)
    table, zeros_out = _build_table(context)
    pooled = _sc_pool(context, table, starts, ends)
    return lax.dynamic_update_slice(zeros_out, pooled, (0, 0, 0))
